# baseline (speedup 1.0000x reference)
import jax
import jax.numpy as jnp
from jax import lax
from jax.experimental import pallas as pl
from jax.experimental.pallas import tpu as pltpu
from jax.experimental.pallas import tpu_sc as plsc

B = 16384
VOCAB = 1000000
D = 32
NCTX = 5
RPL = 128 // D
LINES = VOCAB // RPL

NC = 2
NS = 16
NW = NC * NS
BPW = B // NW
CPW = BPW * NCTX
CHB = 64
CHC = CHB * NCTX
NCH = BPW // CHB
GRP = CHB // 16


def _body(tt_hbm, tidx_hbm, ct_hbm, cidx_hbm, out_hbm,
          tidx_v, cidx_v, tl_v, cl_v, lines_t, lines_c, out_v, sem):
  cid = lax.axis_index("c")
  sid = lax.axis_index("s")
  wid = cid * NS + sid

  pltpu.sync_copy(tidx_hbm.at[pl.ds(wid * BPW, BPW)], tidx_v)
  pltpu.sync_copy(cidx_hbm.at[pl.ds(wid * CPW, CPW)], cidx_v)

  def tline(i, carry):
    tl_v[pl.ds(i * 16, 16)] = lax.shift_right_logical(
        tidx_v[pl.ds(i * 16, 16)], 2)
    return carry

  def cline(i, carry):
    cl_v[pl.ds(i * 16, 16)] = lax.shift_right_logical(
        cidx_v[pl.ds(i * 16, 16)], 2)
    return carry

  lax.fori_loop(0, BPW // 16, tline, 0)
  lax.fori_loop(0, CPW // 16, cline, 0)

  def fire(ch, slot):
    tb = ch * CHB
    cb = ch * CHC
    pltpu.async_copy(tt_hbm.at[tl_v.at[pl.ds(tb, CHB)]],
                     lines_t.at[pl.ds(slot * CHB, CHB)], sem.at[slot])
    pltpu.async_copy(ct_hbm.at[cl_v.at[pl.ds(cb, 128)]],
                     lines_c.at[pl.ds(slot * CHC, 128)], sem.at[slot])
    pltpu.async_copy(ct_hbm.at[cl_v.at[pl.ds(cb + 128, 128)]],
                     lines_c.at[pl.ds(slot * CHC + 128, 128)], sem.at[slot])
    pltpu.async_copy(ct_hbm.at[cl_v.at[pl.ds(cb + 256, 64)]],
                     lines_c.at[pl.ds(slot * CHC + 256, 64)], sem.at[slot])

  def drain(slot):
    pltpu.make_async_copy(tt_hbm.at[pl.ds(0, CHB)],
                          lines_t.at[pl.ds(slot * CHB, CHB)],
                          sem.at[slot]).wait()
    pltpu.make_async_copy(ct_hbm.at[pl.ds(0, CHC)],
                          lines_c.at[pl.ds(slot * CHC, CHC)],
                          sem.at[slot]).wait()

  iota16 = lax.broadcasted_iota(jnp.int32, (16,), 0)
  iota5 = iota16 * NCTX
  zero16 = jnp.zeros((16,), jnp.float32)

  def compute(ch, slot):
    for j in range(GRP):
      lb0 = ch * CHB + j * 16
      tsub = (plsc.load_gather(tidx_v, [lb0 + iota16]) & 3) * D
      trow = slot * CHB + j * 16 + iota16
      accs = []
      crows = []
      csubs = []
      for n in range(NCTX):
        csubs.append(
            (plsc.load_gather(cidx_v, [lb0 * NCTX + n + iota5]) & 3) * D)
        crows.append(slot * CHC + j * 16 * NCTX + n + iota5)
        accs.append(zero16)
      for d in range(D):
        we = plsc.load_gather(lines_t, [trow, tsub + d])
        for n in range(NCTX):
          ce = plsc.load_gather(lines_c, [crows[n], csubs[n] + d])
          accs[n] = accs[n] + ce * we
      for n in range(NCTX):
        out_v[pl.ds(n * BPW + lb0, 16)] = accs[n]

  fire(0, 0)

  def body(ch, carry):
    slot = ch & 1

    @pl.when(ch < NCH - 1)
    def _():
      fire(ch + 1, 1 - slot)

    drain(slot)
    compute(ch, slot)
    return carry

  lax.fori_loop(0, NCH, body, 0)

  pltpu.sync_copy(out_v, out_hbm.at[pl.ds(wid * CPW, CPW)])


@jax.jit
def kernel(target, context, target_table, context_table):
  tidx = target.reshape(B)
  cidx = context.reshape(B * NCTX)
  tt_lines = target_table.reshape(LINES, 128)
  ct_lines = context_table.reshape(LINES, 128)

  mesh = plsc.VectorSubcoreMesh(core_axis_name="c", subcore_axis_name="s")
  run = pl.kernel(
      _body,
      out_type=jax.ShapeDtypeStruct((NW * CPW,), jnp.float32),
      mesh=mesh,
      scratch_types=[
          pltpu.VMEM((BPW,), jnp.int32),
          pltpu.VMEM((CPW,), jnp.int32),
          pltpu.VMEM((BPW,), jnp.int32),
          pltpu.VMEM((CPW,), jnp.int32),
          pltpu.VMEM((2 * CHB, 128), jnp.float32),
          pltpu.VMEM((2 * CHC, 128), jnp.float32),
          pltpu.VMEM((CPW,), jnp.float32),
          pltpu.SemaphoreType.DMA((2,)),
      ],
      compiler_params=pltpu.CompilerParams(needs_layout_passes=False),
  )
  out_flat = run(tt_lines, tidx, ct_lines, cidx)
  return out_flat.reshape(NW, NCTX, BPW).transpose(0, 2, 1).reshape(B, NCTX)
